# baseline (speedup 1.0000x reference)
import functools

import jax
import jax.numpy as jnp
from jax import lax
from jax.experimental import pallas as pl
from jax.experimental.pallas import tpu as pltpu
from jax.experimental.pallas import tpu_sc as plsc

N_NODES = 10000
DH = 128
NB = 1000
EB = 2000
NC = 2
NS = 16
IDXW = 80
HN = 5120
HNP = 5248
F32 = jnp.float32


def _silu_and_d(pre):
    s = jax.nn.sigmoid(pre)
    return pre * s, s * (1.0 + pre * (1.0 - s))


def _row_spec(arr_like, blk):
    shp = arr_like.shape
    if len(shp) == 2:
        return pl.BlockSpec((blk, shp[1]), lambda i: (i, 0))
    return pl.BlockSpec((shp[0], blk, shp[2]), lambda i: (0, i, 0))


def _full_spec(arr_like):
    shp = arr_like.shape
    return pl.BlockSpec(shp, lambda i, _r=len(shp): (0,) * _r)


def _tc_call(body, n_rows, blk, row_in, full_in, out_rowdims, name):
    in_specs = [_row_spec(a, blk) for a in row_in] + [_full_spec(a) for a in full_in]
    out_specs = [pl.BlockSpec((blk, d), lambda i: (i, 0)) for d in out_rowdims]
    out_shape = [jax.ShapeDtypeStruct((n_rows, d), F32) for d in out_rowdims]
    return pl.pallas_call(
        body,
        grid=(n_rows // blk,),
        in_specs=in_specs,
        out_specs=out_specs,
        out_shape=out_shape,
        name=name,
    )(*row_in, *full_in)


BF16 = jnp.bfloat16


def _mm(a, b):
    return jnp.dot(a.astype(BF16), b.astype(BF16), preferred_element_type=F32)


def _r16(a):
    return a.astype(BF16).astype(F32)


def _sc_gather2(table0, table1, idx0, idx1, D):
    E = idx0.shape[0]
    w = IDXW if D <= 128 else IDXW // 2
    nj = 5
    ce = nj * w
    pairs = E // NS // (2 * ce)
    per_e = E // NS
    mesh = plsc.VectorSubcoreMesh(core_axis_name="c", subcore_axis_name="s")

    @functools.partial(
        pl.kernel,
        out_type=(
            jax.ShapeDtypeStruct((E, D), F32),
            jax.ShapeDtypeStruct((E, D), F32),
        ),
        mesh=mesh,
        scratch_types=[
            [[pltpu.VMEM((w,), jnp.int32) for _ in range(nj)] for _ in range(2)],
            [pltpu.VMEM((ce, D), F32) for _ in range(2)],
            pltpu.SemaphoreType.DMA,
            pltpu.SemaphoreType.DMA,
            pltpu.SemaphoreType.DMA,
        ],
    )
    def k(t0, t1, i0, i1, o0, o1, idx_b, rows_b, sem_i, sem_g, sem_o):
        cid = lax.axis_index("c")
        sid = lax.axis_index("s")

        def stream(t, idx, out):
            def pair_body(p, carry):
                b0 = sid * per_e + 2 * p * ce
                b1 = b0 + ce
                i0s = [pltpu.async_copy(idx.at[pl.ds(b0 + j * w, w)],
                                        idx_b[0][j], sem_i) for j in range(nj)]
                i1s = [pltpu.async_copy(idx.at[pl.ds(b1 + j * w, w)],
                                        idx_b[1][j], sem_i) for j in range(nj)]
                for cp in i0s:
                    cp.wait()
                g0s = [pltpu.async_copy(t.at[idx_b[0][j]],
                                        rows_b[0].at[pl.ds(j * w, w)], sem_g)
                       for j in range(nj)]
                for cp in i1s:
                    cp.wait()
                for cp in g0s:
                    cp.wait()
                o0c = pltpu.async_copy(rows_b[0], out.at[pl.ds(b0, ce)], sem_o)
                g1s = [pltpu.async_copy(t.at[idx_b[1][j]],
                                        rows_b[1].at[pl.ds(j * w, w)], sem_g)
                       for j in range(nj)]
                for cp in g1s:
                    cp.wait()
                o0c.wait()
                pltpu.sync_copy(rows_b[1], out.at[pl.ds(b1, ce)])
                return carry

            lax.fori_loop(0, pairs, pair_body, 0)

        @pl.when(cid == 0)
        def _():
            stream(t0, i0, o0)

        @pl.when(cid != 0)
        def _():
            stream(t1, i1, o1)

    return k(table0, table1, idx0, idx1)


def _sc_scatter_rs(vals, idx_lo, idx_hi):
    E = idx_lo.shape[0]
    w = IDXW // 2
    nj = 5
    ce = nj * w
    pairs = E // NS // (2 * ce)
    per_e = E // NS
    per_sub = HNP // NS
    mesh = plsc.VectorSubcoreMesh(core_axis_name="c", subcore_axis_name="s")
    zeros = jnp.zeros((per_sub, DH), F32)

    @functools.partial(
        pl.kernel,
        out_type=jax.ShapeDtypeStruct((NC, HNP, DH), F32),
        mesh=mesh,
        scratch_types=[
            [[pltpu.VMEM((w,), jnp.int32) for _ in range(nj)] for _ in range(2)],
            [pltpu.VMEM((ce, DH), F32) for _ in range(2)],
            pltpu.VMEM_SHARED((HNP, DH), F32),
            pltpu.SemaphoreType.DMA,
            pltpu.SemaphoreType.DMA,
            pltpu.SemaphoreType.DMA,
        ],
    )
    def k(v, z, ilo, ihi, out, idx_b, rows_b, acc, sem_i, sem_v, sem_s):
        cid = lax.axis_index("c")
        sid = lax.axis_index("s")
        ob = sid * per_sub
        pltpu.sync_copy(z, acc.at[pl.ds(ob, per_sub)])
        plsc.subcore_barrier()

        def stream(idx):
            def pair_body(p, carry):
                b0 = sid * per_e + 2 * p * ce
                b1 = b0 + ce
                v0c = pltpu.async_copy(v.at[pl.ds(b0, ce)], rows_b[0], sem_v)
                i0s = [pltpu.async_copy(idx.at[pl.ds(b0 + j * w, w)],
                                        idx_b[0][j], sem_i) for j in range(nj)]
                v1c = pltpu.async_copy(v.at[pl.ds(b1, ce)], rows_b[1], sem_v)
                i1s = [pltpu.async_copy(idx.at[pl.ds(b1 + j * w, w)],
                                        idx_b[1][j], sem_i) for j in range(nj)]
                v0c.wait()
                for cp in i0s:
                    cp.wait()
                a0s = [pltpu.async_copy(rows_b[0].at[pl.ds(j * w, w)],
                                        acc.at[idx_b[0][j]], sem_s, add=True)
                       for j in range(nj)]
                v1c.wait()
                for cp in i1s:
                    cp.wait()
                for cp in a0s:
                    cp.wait()
                a1s = [pltpu.async_copy(rows_b[1].at[pl.ds(j * w, w)],
                                        acc.at[idx_b[1][j]], sem_s, add=True)
                       for j in range(nj)]
                for cp in a1s:
                    cp.wait()
                return carry

            lax.fori_loop(0, pairs, pair_body, 0)

        @pl.when(cid == 0)
        def _():
            stream(ilo)

        @pl.when(cid != 0)
        def _():
            stream(ihi)

        plsc.subcore_barrier()
        pltpu.sync_copy(acc.at[pl.ds(ob, per_sub)],
                        out.at[cid].at[pl.ds(ob, per_sub)])

    return k(vals, zeros, idx_lo, idx_hi)


def _segsum(vals, idx_lo, idx_hi, n):
    out = _sc_scatter_rs(vals, idx_lo, idx_hi)
    return jnp.concatenate([out[0, :HN], out[1, :n - HN]], axis=0)


def _enc_node_body(z, n, q16, W0z, W0n, b0, W1, b1, A0, B0,
                   xenc_o, sd_o, ts_o, td_o):
    pre = _mm(z[...], W0z[...]) + _mm(n[...], W0n[...]) + b0[...]
    h, sd = _silu_and_d(pre)
    sd_o[...] = sd
    xe = _mm(h, W1[...]) + b1[...]
    xenc_o[...] = xe
    qv = q16[...]
    pad = jnp.zeros((qv.shape[0], 112), F32)
    ts_o[...] = jnp.concatenate([_mm(xe, A0[...]), qv, pad], axis=1)
    td_o[...] = jnp.concatenate([_mm(xe, B0[...]), qv, pad], axis=1)


def _edge_fwd0_body(gs, gd, V0p, v0n, c0, V1, c1, C0, be0, W1, be1,
                    ea_o, sd_o, e1_o):
    d16 = gs[:, DH:DH + 16] - gd[:, DH:DH + 16]
    nrm = jnp.sqrt(jnp.sum(d16 * d16, axis=1, keepdims=True))
    pre0 = _mm(d16, V0p[...]) + _r16(nrm) * _r16(v0n[...]) + c0[...]
    h0, _ = _silu_and_d(pre0)
    e0 = _mm(h0, V1[...]) + c1[...]
    pre = _mm(e0, C0[...]) + gs[:, :DH] + gd[:, :DH] + be0[...]
    h, sd = _silu_and_d(pre)
    sd_o[...] = sd
    ea = _mm(h, W1[...]) + be1[...]
    ea_o[...] = ea
    e1_o[...] = e0 + ea


def _edge_fwd1_body(e, gs, gd, C, be0, W1, be1, ea_o, sd_o):
    pre = _mm(e[...], C[...]) + gs[...] + gd[...] + be0[...]
    h, sd = _silu_and_d(pre)
    sd_o[...] = sd
    ea_o[...] = _mm(h, W1[...]) + be1[...]


def _node_fwd_body(with_tables, x, agg, Wa, Wb, bn0, W1, bn1, *rest):
    if with_tables:
        A, B, xn_o, sd_o, xs_o, xd_o = rest
    else:
        xn_o, sd_o = rest
    pre = _mm(x[...], Wa[...]) + _mm(agg[...], Wb[...]) + bn0[...]
    h, sd = _silu_and_d(pre)
    sd_o[...] = sd
    xn = x[...] + _mm(h, W1[...]) + bn1[...]
    xn_o[...] = xn
    if with_tables:
        xs_o[...] = _mm(xn, A[...])
        xd_o[...] = _mm(xn, B[...])


_MMT_TERMS = None


def _mmt_terms():
    global _MMT_TERMS
    if _MMT_TERMS is None:
        pos = {}
        j = 0
        for a in range(6):
            for b in range(a + 1):
                pos[(a, b)] = j
                j += 1
        terms = []
        for a in range(6):
            for b in range(6):
                t = [(pos[(a, c)], pos[(b, c)]) for c in range(min(a, b) + 1)]
                terms.append(t)
        _MMT_TERMS = terms
    return _MMT_TERMS


def _dec_body(x, WE0, bE0, wE1, bE1, WS0, bS0, wS1, bS1,
              WL0, bL0, WL1f, bL1f, WM0, bM0, WM1, bM1,
              WE0T, WS0T,
              es_o, gxE_o, gxS_o, lcols_o, mmt_o):
    xv = x[...]
    preE = _mm(xv, WE0[...]) + bE0[...]
    hE, sdE = _silu_and_d(preE)
    Ev = jnp.sum(_r16(hE) * _r16(wE1[...]), axis=1, keepdims=True) + bE1[...]
    gxE_o[...] = _mm(sdE * _r16(wE1[...]), WE0T[...])
    preS = _mm(xv, WS0[...]) + bS0[...]
    hS, sdS = _silu_and_d(preS)
    Sv = jnp.sum(_r16(hS) * _r16(wS1[...]), axis=1, keepdims=True) + bS1[...]
    gxS_o[...] = _mm(sdS * _r16(wS1[...]), WS0T[...])
    z1 = jnp.zeros_like(Ev)
    es_o[...] = jnp.concatenate([Ev, Sv, z1, z1, z1, z1, z1, z1], axis=1)
    preL = _mm(xv, WL0[...]) + bL0[...]
    hL, _ = _silu_and_d(preL)
    lcols_o[...] = _mm(hL, WL1f[...]) + bL1f[...]
    preM = _mm(xv, WM0[...]) + bM0[...]
    hM, _ = _silu_and_d(preM)
    m = _r16(_mm(hM, WM1[...]) + bM1[...])
    cols = []
    for t in _mmt_terms():
        acc = m[:, t[0][0]:t[0][0] + 1] * m[:, t[0][1]:t[0][1] + 1]
        for (pa, pb) in t[1:]:
            acc = acc + m[:, pa:pa + 1] * m[:, pb:pb + 1]
        cols.append(acc)
    mmt_o[...] = jnp.concatenate(cols, axis=1)


def _node_bwd1_body(gE, gS, sdn, Wn1T, WaT, WbT,
                    gaE_o, gaS_o, gpE_o, gpS_o):
    for g, ga_o, gp_o in ((gE, gaE_o, gpE_o), (gS, gaS_o, gpS_o)):
        gpren = _mm(g[...], Wn1T[...]) * sdn[...]
        ga_o[...] = _mm(gpren, WbT[...])
        gp_o[...] = g[...] + _mm(gpren, WaT[...])


def _edge_bwd1_body(ggE, ggS, sd, We1T, CT, AT, BT,
                    gaE_o, gbE_o, gaS_o, gbS_o, geE_o, geS_o):
    for gg, ga_o, gb_o, ge_o in ((ggE, gaE_o, gbE_o, geE_o),
                                 (ggS, gaS_o, gbS_o, geS_o)):
        gpr = _mm(gg[...], We1T[...]) * sd[...]
        ga_o[...] = _mm(gpr, AT[...])
        gb_o[...] = _mm(gpr, BT[...])
        ge_o[...] = _mm(gpr, CT[...])


def _edge_bwd0_body(ggE, ggS, geE, geS, sd, We1T, AT, BT,
                    gaE_o, gbE_o, gaS_o, gbS_o):
    for gg, ge, ga_o, gb_o in ((ggE, geE, gaE_o, gbE_o),
                               (ggS, geS, gaS_o, gbS_o)):
        gpr = _mm(gg[...] + ge[...], We1T[...]) * sd[...]
        ga_o[...] = _mm(gpr, AT[...])
        gb_o[...] = _mm(gpr, BT[...])


def _node_bwd_mid_body(gpE, gpS, saE, sbE, saS, sbS, sdn0,
                       Wn1T0, WaT0, WbT0,
                       gaE_o, gaS_o, gpE_o, gpS_o):
    for gp, sa, sb, ga_o, gp_o in (
            (gpE, saE, sbE, gaE_o, gpE_o), (gpS, saS, sbS, gaS_o, gpS_o)):
        g1 = gp[...] + sa[...] + sb[...]
        gpren = _mm(g1, Wn1T0[...]) * sdn0[...]
        ga_o[...] = _mm(gpren, WbT0[...])
        gp_o[...] = g1 + _mm(gpren, WaT0[...])


def _node_bwd_final_body(gpE, gpS, saE, sbE, saS, sbS, sdenc,
                         W1eT, W0eT, dzE_o, dzS_o):
    for gp, sa, sb, dz_o in (
            (gpE, saE, sbE, dzE_o), (gpS, saS, sbS, dzS_o)):
        g0 = gp[...] + sa[...] + sb[...]
        gpre = _mm(g0, W1eT[...]) * sdenc[...]
        dz_o[...] = _mm(gpre, W0eT[...])


def kernel(z, n, edge_index, q_0, params):
    N = z.shape[0]
    E = edge_index.shape[1]
    src1 = edge_index[0]
    dest1 = edge_index[1]
    s_lo = jnp.where(src1 < HN, src1, HN)
    s_hi = jnp.where(src1 >= HN, src1 - HN, HN)
    d_lo = jnp.where(dest1 < HN, dest1, HN)
    d_hi = jnp.where(dest1 >= HN, dest1 - HN, HN)

    (W0e, b0e), (W1e, b1e) = params["enc_node"]
    (V0, c0), (V1, c1) = params["enc_edge"]
    em = params["edge_mlps"]
    nm = params["node_mlps"]
    C = [em[p][0][0][:DH] for p in range(2)]
    A = [em[p][0][0][DH:2 * DH] for p in range(2)]
    B = [em[p][0][0][2 * DH:] for p in range(2)]
    be0 = [em[p][0][1].reshape(1, -1) for p in range(2)]
    We1 = [em[p][1][0] for p in range(2)]
    be1 = [em[p][1][1].reshape(1, -1) for p in range(2)]
    Wa = [nm[p][0][0][:DH] for p in range(2)]
    Wb = [nm[p][0][0][DH:] for p in range(2)]
    bn0 = [nm[p][0][1].reshape(1, -1) for p in range(2)]
    Wn1 = [nm[p][1][0] for p in range(2)]
    bn1 = [nm[p][1][1].reshape(1, -1) for p in range(2)]

    q16 = jnp.pad(q_0, ((0, 0), (0, 13)))
    xenc, sdenc, tbl_s, tbl_d = _tc_call(
        _enc_node_body, N, NB,
        [z, n, q16],
        [W0e[:6], W0e[6:], b0e.reshape(1, -1), W1e, b1e.reshape(1, -1),
         A[0], B[0]],
        [DH, DH, 256, 256], "enc_node")

    gs0, gd0 = _sc_gather2(tbl_s, tbl_d, src1, dest1, 256)

    V0p = jnp.pad(V0[:3], ((0, 13), (0, 0)))
    ea0, sde0, e1 = _tc_call(
        _edge_fwd0_body, E, EB,
        [gs0, gd0],
        [V0p, V0[3].reshape(1, -1), c0.reshape(1, -1), V1, c1.reshape(1, -1),
         C[0], be0[0], We1[0], be1[0]],
        [DH, DH, DH], "edge_fwd0")

    agg0 = _segsum(ea0, d_lo, d_hi, N)
    x1, sdn0, xs1, xd1 = _tc_call(
        functools.partial(_node_fwd_body, True), N, NB,
        [xenc, agg0],
        [Wa[0], Wb[0], bn0[0], Wn1[0], bn1[0], A[1], B[1]],
        [DH, DH, DH, DH], "node_fwd0")

    gxs1, gxd1 = _sc_gather2(xs1, xd1, src1, dest1, DH)
    ea1, sde1 = _tc_call(
        _edge_fwd1_body, E, EB,
        [e1, gxs1, gxd1],
        [C[1], be0[1], We1[1], be1[1]],
        [DH, DH], "edge_fwd1")

    agg1 = _segsum(ea1, d_lo, d_hi, N)
    x2, sdn1 = _tc_call(
        functools.partial(_node_fwd_body, False), N, NB,
        [x1, agg1],
        [Wa[1], Wb[1], bn0[1], Wn1[1], bn1[1]],
        [DH, DH], "node_fwd1")

    (WE0, bE0), (WE1, bE1) = params["dec_E"]
    (WS0, bS0), (WS1, bS1) = params["dec_S"]
    (WL0, bL0), (WL1, bL1) = params["dec_L"]
    (WM0, bM0), (WM1, bM1) = params["dec_M"]

    import numpy as _np
    rL, cL = _np.tril_indices(6, -1)
    PL = _np.zeros((15, 36), _np.float32)
    for j, (a, b) in enumerate(zip(rL, cL)):
        PL[j, a * 6 + b] = 1.0
        PL[j, b * 6 + a] = -1.0
    PLj = jnp.asarray(PL)
    WL1f = WL1 @ PLj
    bL1f = (bL1 @ PLj).reshape(1, -1)

    es8, gxE, gxS, lcols, mmt = _tc_call(
        _dec_body, N, NB,
        [x2],
        [WE0, bE0.reshape(1, -1), WE1[:, 0].reshape(1, -1), bE1.reshape(1, -1),
         WS0, bS0.reshape(1, -1), WS1[:, 0].reshape(1, -1), bS1.reshape(1, -1),
         WL0, bL0.reshape(1, -1), WL1f, bL1f,
         WM0, bM0.reshape(1, -1), WM1, bM1.reshape(1, -1),
         WE0.T, WS0.T],
        [8, DH, DH, 36, 36], "dec")

    gaE1, gaS1, gpE1, gpS1 = _tc_call(
        _node_bwd1_body, N, NB,
        [gxE, gxS, sdn1],
        [Wn1[1].T, Wa[1].T, Wb[1].T],
        [DH, DH, DH, DH], "node_bwd1")

    ggE1, ggS1 = _sc_gather2(gaE1, gaS1, dest1, dest1, DH)
    gAE1, gBE1, gAS1, gBS1, geE, geS = _tc_call(
        _edge_bwd1_body, E, EB,
        [ggE1, ggS1, sde1],
        [We1[1].T, C[1].T, A[1].T, B[1].T],
        [DH, DH, DH, DH, DH, DH], "edge_bwd1")

    saE1 = _segsum(gAE1, s_lo, s_hi, N)
    sbE1 = _segsum(gBE1, d_lo, d_hi, N)
    saS1 = _segsum(gAS1, s_lo, s_hi, N)
    sbS1 = _segsum(gBS1, d_lo, d_hi, N)

    gaE0, gaS0, gpE0, gpS0 = _tc_call(
        _node_bwd_mid_body, N, NB,
        [gpE1, gpS1, saE1, sbE1, saS1, sbS1, sdn0],
        [Wn1[0].T, Wa[0].T, Wb[0].T],
        [DH, DH, DH, DH], "node_bwd_mid")

    ggE0, ggS0 = _sc_gather2(gaE0, gaS0, dest1, dest1, DH)
    gAE0, gBE0, gAS0, gBS0 = _tc_call(
        _edge_bwd0_body, E, EB,
        [ggE0, ggS0, geE, geS, sde0],
        [We1[0].T, A[0].T, B[0].T],
        [DH, DH, DH, DH], "edge_bwd0")

    saE0 = _segsum(gAE0, s_lo, s_hi, N)
    sbE0 = _segsum(gBE0, d_lo, d_hi, N)
    saS0 = _segsum(gAS0, s_lo, s_hi, N)
    sbS0 = _segsum(gBS0, d_lo, d_hi, N)

    dzE8, dzS8 = _tc_call(
        _node_bwd_final_body, N, NB,
        [gpE0, gpS0, saE0, sbE0, saS0, sbS0, sdenc],
        [W1e.T, W0e.T],
        [8, 8], "node_bwd_final")

    L = lcols.reshape(N, 6, 6)
    M = mmt.reshape(N, 6, 6)
    dEdz = dzE8[:, :6].reshape(N, 6, 1)
    dSdz = dzS8[:, :6].reshape(N, 6, 1)
    E_out = es8[:, 0:1]
    S_out = es8[:, 1:2]
    return (L, M, dEdz, dSdz, E_out, S_out)


# speedup vs baseline: 1.0389x; 1.0389x over previous
import functools

import jax
import jax.numpy as jnp
from jax import lax
from jax.experimental import pallas as pl
from jax.experimental.pallas import tpu as pltpu
from jax.experimental.pallas import tpu_sc as plsc

N_NODES = 10000
DH = 128
NB = 1000
EB = 2000
NC = 2
NS = 16
IDXW = 80
HN = 5120
HNP = 5248
F32 = jnp.float32


def _silu_and_d(pre):
    s = jax.nn.sigmoid(pre)
    return pre * s, s * (1.0 + pre * (1.0 - s))


def _row_spec(arr_like, blk):
    shp = arr_like.shape
    if len(shp) == 2:
        return pl.BlockSpec((blk, shp[1]), lambda i: (i, 0))
    return pl.BlockSpec((shp[0], blk, shp[2]), lambda i: (0, i, 0))


def _full_spec(arr_like):
    shp = arr_like.shape
    return pl.BlockSpec(shp, lambda i, _r=len(shp): (0,) * _r)


def _tc_call(body, n_rows, blk, row_in, full_in, out_rowdims, name):
    in_specs = [_row_spec(a, blk) for a in row_in] + [_full_spec(a) for a in full_in]
    out_specs = [pl.BlockSpec((blk, d), lambda i: (i, 0)) for d in out_rowdims]
    out_shape = [jax.ShapeDtypeStruct((n_rows, d), F32) for d in out_rowdims]
    return pl.pallas_call(
        body,
        grid=(n_rows // blk,),
        in_specs=in_specs,
        out_specs=out_specs,
        out_shape=out_shape,
        name=name,
    )(*row_in, *full_in)


BF16 = jnp.bfloat16


def _mm(a, b):
    return jnp.dot(a.astype(BF16), b.astype(BF16), preferred_element_type=F32)


def _r16(a):
    return a.astype(BF16).astype(F32)


def _sc_gather2(table0, table1, idx0, idx1, D):
    E = idx0.shape[0]
    w = IDXW if D <= 128 else IDXW // 2
    nj = 5
    ce = nj * w
    pairs = E // NS // (2 * ce)
    per_e = E // NS
    mesh = plsc.VectorSubcoreMesh(core_axis_name="c", subcore_axis_name="s")

    @functools.partial(
        pl.kernel,
        out_type=(
            jax.ShapeDtypeStruct((E, D), F32),
            jax.ShapeDtypeStruct((E, D), F32),
        ),
        mesh=mesh,
        scratch_types=[
            [[pltpu.VMEM((w,), jnp.int32) for _ in range(nj)] for _ in range(2)],
            [pltpu.VMEM((ce, D), F32) for _ in range(2)],
            pltpu.SemaphoreType.DMA,
            pltpu.SemaphoreType.DMA,
            pltpu.SemaphoreType.DMA,
        ],
    )
    def k(t0, t1, i0, i1, o0, o1, idx_b, rows_b, sem_i, sem_g, sem_o):
        cid = lax.axis_index("c")
        sid = lax.axis_index("s")

        def stream(t, idx, out):
            def pair_body(p, carry):
                b0 = sid * per_e + 2 * p * ce
                b1 = b0 + ce
                i0s = [pltpu.async_copy(idx.at[pl.ds(b0 + j * w, w)],
                                        idx_b[0][j], sem_i) for j in range(nj)]
                i1s = [pltpu.async_copy(idx.at[pl.ds(b1 + j * w, w)],
                                        idx_b[1][j], sem_i) for j in range(nj)]
                for cp in i0s:
                    cp.wait()
                g0s = [pltpu.async_copy(t.at[idx_b[0][j]],
                                        rows_b[0].at[pl.ds(j * w, w)], sem_g)
                       for j in range(nj)]
                for cp in i1s:
                    cp.wait()
                for cp in g0s:
                    cp.wait()
                o0c = pltpu.async_copy(rows_b[0], out.at[pl.ds(b0, ce)], sem_o)
                g1s = [pltpu.async_copy(t.at[idx_b[1][j]],
                                        rows_b[1].at[pl.ds(j * w, w)], sem_g)
                       for j in range(nj)]
                for cp in g1s:
                    cp.wait()
                o0c.wait()
                pltpu.sync_copy(rows_b[1], out.at[pl.ds(b1, ce)])
                return carry

            lax.fori_loop(0, pairs, pair_body, 0)

        @pl.when(cid == 0)
        def _():
            stream(t0, i0, o0)

        @pl.when(cid != 0)
        def _():
            stream(t1, i1, o1)

    return k(table0, table1, idx0, idx1)


def _sc_scatter_rs(vals, idx_lo, idx_hi):
    E = idx_lo.shape[0]
    w = IDXW // 2
    nj = 5
    ce = nj * w
    pairs = E // NS // (2 * ce)
    per_e = E // NS
    per_sub = HNP // NS
    mesh = plsc.VectorSubcoreMesh(core_axis_name="c", subcore_axis_name="s")
    zeros = jnp.zeros((per_sub, DH), F32)

    @functools.partial(
        pl.kernel,
        out_type=jax.ShapeDtypeStruct((NC, HNP, DH), F32),
        mesh=mesh,
        scratch_types=[
            [[pltpu.VMEM((w,), jnp.int32) for _ in range(nj)] for _ in range(2)],
            [pltpu.VMEM((ce, DH), F32) for _ in range(2)],
            pltpu.VMEM_SHARED((HNP, DH), F32),
            pltpu.SemaphoreType.DMA,
            pltpu.SemaphoreType.DMA,
            pltpu.SemaphoreType.DMA,
        ],
    )
    def k(v, z, ilo, ihi, out, idx_b, rows_b, acc, sem_i, sem_v, sem_s):
        cid = lax.axis_index("c")
        sid = lax.axis_index("s")
        ob = sid * per_sub
        pltpu.sync_copy(z, acc.at[pl.ds(ob, per_sub)])
        plsc.subcore_barrier()

        def stream(idx):
            def pair_body(p, carry):
                b0 = sid * per_e + 2 * p * ce
                b1 = b0 + ce
                v0c = pltpu.async_copy(v.at[pl.ds(b0, ce)], rows_b[0], sem_v)
                i0s = [pltpu.async_copy(idx.at[pl.ds(b0 + j * w, w)],
                                        idx_b[0][j], sem_i) for j in range(nj)]
                v1c = pltpu.async_copy(v.at[pl.ds(b1, ce)], rows_b[1], sem_v)
                i1s = [pltpu.async_copy(idx.at[pl.ds(b1 + j * w, w)],
                                        idx_b[1][j], sem_i) for j in range(nj)]
                v0c.wait()
                for cp in i0s:
                    cp.wait()
                a0s = [pltpu.async_copy(rows_b[0].at[pl.ds(j * w, w)],
                                        acc.at[idx_b[0][j]], sem_s, add=True)
                       for j in range(nj)]
                v1c.wait()
                for cp in i1s:
                    cp.wait()
                for cp in a0s:
                    cp.wait()
                a1s = [pltpu.async_copy(rows_b[1].at[pl.ds(j * w, w)],
                                        acc.at[idx_b[1][j]], sem_s, add=True)
                       for j in range(nj)]
                for cp in a1s:
                    cp.wait()
                return carry

            lax.fori_loop(0, pairs, pair_body, 0)

        @pl.when(cid == 0)
        def _():
            stream(ilo)

        @pl.when(cid != 0)
        def _():
            stream(ihi)

        plsc.subcore_barrier()
        pltpu.sync_copy(acc.at[pl.ds(ob, per_sub)],
                        out.at[cid].at[pl.ds(ob, per_sub)])

    return k(vals, zeros, idx_lo, idx_hi)


def _segsum(vals, idx_lo, idx_hi, n):
    out = _sc_scatter_rs(vals, idx_lo, idx_hi)
    return jnp.concatenate([out[0, :HN], out[1, :n - HN]], axis=0)


def _enc_node_body(z, n, q16, W0z, W0n, b0, W1, b1, A0, B0,
                   xenc_o, sd_o, ts_o, td_o):
    pre = _mm(z[...], W0z[...]) + _mm(n[...], W0n[...]) + b0[...]
    h, sd = _silu_and_d(pre)
    sd_o[...] = sd
    xe = _mm(h, W1[...]) + b1[...]
    xenc_o[...] = xe
    qv = q16[...]
    pad = jnp.zeros((qv.shape[0], 112), F32)
    ts_o[...] = jnp.concatenate([_mm(xe, A0[...]), qv, pad], axis=1)
    td_o[...] = jnp.concatenate([_mm(xe, B0[...]), qv, pad], axis=1)


def _edge_fwd0_body(gs, gd, V0p, v0n, c0, V1, c1, C0, be0, W1, be1,
                    ea_o, sd_o, e1_o):
    d16 = gs[:, DH:DH + 16] - gd[:, DH:DH + 16]
    nrm = jnp.sqrt(jnp.sum(d16 * d16, axis=1, keepdims=True))
    pre0 = _mm(d16, V0p[...]) + _r16(nrm) * _r16(v0n[...]) + c0[...]
    h0, _ = _silu_and_d(pre0)
    e0 = _mm(h0, V1[...]) + c1[...]
    pre = _mm(e0, C0[...]) + gs[:, :DH] + gd[:, :DH] + be0[...]
    h, sd = _silu_and_d(pre)
    sd_o[...] = sd
    ea = _mm(h, W1[...]) + be1[...]
    ea_o[...] = ea
    e1_o[...] = e0 + ea


def _edge_fwd1_body(e, gs, gd, C, be0, W1, be1, ea_o, sd_o):
    pre = _mm(e[...], C[...]) + gs[...] + gd[...] + be0[...]
    h, sd = _silu_and_d(pre)
    sd_o[...] = sd
    ea_o[...] = _mm(h, W1[...]) + be1[...]


def _node_fwd_body(with_tables, x, agg, Wa, Wb, bn0, W1, bn1, *rest):
    if with_tables:
        A, B, xn_o, sd_o, xs_o, xd_o = rest
    else:
        xn_o, sd_o = rest
    pre = _mm(x[...], Wa[...]) + _mm(agg[...], Wb[...]) + bn0[...]
    h, sd = _silu_and_d(pre)
    sd_o[...] = sd
    xn = x[...] + _mm(h, W1[...]) + bn1[...]
    xn_o[...] = xn
    if with_tables:
        xs_o[...] = _mm(xn, A[...])
        xd_o[...] = _mm(xn, B[...])


_MMT_TERMS = None


def _mmt_terms():
    global _MMT_TERMS
    if _MMT_TERMS is None:
        pos = {}
        j = 0
        for a in range(6):
            for b in range(a + 1):
                pos[(a, b)] = j
                j += 1
        terms = []
        for a in range(6):
            for b in range(6):
                t = [(pos[(a, c)], pos[(b, c)]) for c in range(min(a, b) + 1)]
                terms.append(t)
        _MMT_TERMS = terms
    return _MMT_TERMS


def _dec_body(x, WE0, bE0, wE1, bE1, WS0, bS0, wS1, bS1,
              WL0, bL0, WL1f, bL1f, WM0, bM0, WM1, bM1,
              WE0T, WS0T,
              es_o, gxE_o, gxS_o, lcols_o, mmt_o):
    xv = x[...]
    preE = _mm(xv, WE0[...]) + bE0[...]
    hE, sdE = _silu_and_d(preE)
    Ev = jnp.sum(_r16(hE) * _r16(wE1[...]), axis=1, keepdims=True) + bE1[...]
    gxE_o[...] = _mm(sdE * _r16(wE1[...]), WE0T[...])
    preS = _mm(xv, WS0[...]) + bS0[...]
    hS, sdS = _silu_and_d(preS)
    Sv = jnp.sum(_r16(hS) * _r16(wS1[...]), axis=1, keepdims=True) + bS1[...]
    gxS_o[...] = _mm(sdS * _r16(wS1[...]), WS0T[...])
    z1 = jnp.zeros_like(Ev)
    es_o[...] = jnp.concatenate([Ev, Sv, z1, z1, z1, z1, z1, z1], axis=1)
    preL = _mm(xv, WL0[...]) + bL0[...]
    hL, _ = _silu_and_d(preL)
    lcols_o[...] = _mm(hL, WL1f[...]) + bL1f[...]
    preM = _mm(xv, WM0[...]) + bM0[...]
    hM, _ = _silu_and_d(preM)
    m = _r16(_mm(hM, WM1[...]) + bM1[...])
    cols = []
    for t in _mmt_terms():
        acc = m[:, t[0][0]:t[0][0] + 1] * m[:, t[0][1]:t[0][1] + 1]
        for (pa, pb) in t[1:]:
            acc = acc + m[:, pa:pa + 1] * m[:, pb:pb + 1]
        cols.append(acc)
    mmt_o[...] = jnp.concatenate(cols, axis=1)


def _node_bwd1_body(gE, gS, sdn, Wn1T, WaT, WbT,
                    gaE_o, gaS_o, gpE_o, gpS_o):
    for g, ga_o, gp_o in ((gE, gaE_o, gpE_o), (gS, gaS_o, gpS_o)):
        gpren = _mm(g[...], Wn1T[...]) * sdn[...]
        ga_o[...] = _mm(gpren, WbT[...])
        gp_o[...] = g[...] + _mm(gpren, WaT[...])


def _edge_bwd1_body(ggE, ggS, sd, We1T, CT,
                    gprE_o, gprS_o, geE_o, geS_o):
    for gg, gpr_o, ge_o in ((ggE, gprE_o, geE_o), (ggS, gprS_o, geS_o)):
        gpr = _mm(gg[...], We1T[...]) * sd[...]
        gpr_o[...] = gpr
        ge_o[...] = _mm(gpr, CT[...])


def _edge_bwd0_body(ggE, ggS, geE, geS, sd, We1T, gprE_o, gprS_o):
    for gg, ge, gpr_o in ((ggE, geE, gprE_o), (ggS, geS, gprS_o)):
        gpr_o[...] = _mm(gg[...] + ge[...], We1T[...]) * sd[...]


def _node_bwd_mid_body(gpE, gpS, ssE, sdE, ssS, sdS, sdn0,
                       A1T, B1T, Wn1T0, WaT0, WbT0,
                       gaE_o, gaS_o, gpE_o, gpS_o):
    for gp, ss, sdst, ga_o, gp_o in (
            (gpE, ssE, sdE, gaE_o, gpE_o), (gpS, ssS, sdS, gaS_o, gpS_o)):
        g1 = gp[...] + _mm(ss[...], A1T[...]) + _mm(sdst[...], B1T[...])
        gpren = _mm(g1, Wn1T0[...]) * sdn0[...]
        ga_o[...] = _mm(gpren, WbT0[...])
        gp_o[...] = g1 + _mm(gpren, WaT0[...])


def _node_bwd_final_body(gpE, gpS, ssE, sdE, ssS, sdS, sdenc,
                         A0T, B0T, W1eT, W0eT, dzE_o, dzS_o):
    for gp, ss, sdst, dz_o in (
            (gpE, ssE, sdE, dzE_o), (gpS, ssS, sdS, dzS_o)):
        g0 = gp[...] + _mm(ss[...], A0T[...]) + _mm(sdst[...], B0T[...])
        gpre = _mm(g0, W1eT[...]) * sdenc[...]
        dz_o[...] = _mm(gpre, W0eT[...])


def kernel(z, n, edge_index, q_0, params):
    N = z.shape[0]
    E = edge_index.shape[1]
    src1 = edge_index[0]
    dest1 = edge_index[1]
    s_lo = jnp.where(src1 < HN, src1, HN)
    s_hi = jnp.where(src1 >= HN, src1 - HN, HN)
    d_lo = jnp.where(dest1 < HN, dest1, HN)
    d_hi = jnp.where(dest1 >= HN, dest1 - HN, HN)

    (W0e, b0e), (W1e, b1e) = params["enc_node"]
    (V0, c0), (V1, c1) = params["enc_edge"]
    em = params["edge_mlps"]
    nm = params["node_mlps"]
    C = [em[p][0][0][:DH] for p in range(2)]
    A = [em[p][0][0][DH:2 * DH] for p in range(2)]
    B = [em[p][0][0][2 * DH:] for p in range(2)]
    be0 = [em[p][0][1].reshape(1, -1) for p in range(2)]
    We1 = [em[p][1][0] for p in range(2)]
    be1 = [em[p][1][1].reshape(1, -1) for p in range(2)]
    Wa = [nm[p][0][0][:DH] for p in range(2)]
    Wb = [nm[p][0][0][DH:] for p in range(2)]
    bn0 = [nm[p][0][1].reshape(1, -1) for p in range(2)]
    Wn1 = [nm[p][1][0] for p in range(2)]
    bn1 = [nm[p][1][1].reshape(1, -1) for p in range(2)]

    q16 = jnp.pad(q_0, ((0, 0), (0, 13)))
    xenc, sdenc, tbl_s, tbl_d = _tc_call(
        _enc_node_body, N, NB,
        [z, n, q16],
        [W0e[:6], W0e[6:], b0e.reshape(1, -1), W1e, b1e.reshape(1, -1),
         A[0], B[0]],
        [DH, DH, 256, 256], "enc_node")

    gs0, gd0 = _sc_gather2(tbl_s, tbl_d, src1, dest1, 256)

    V0p = jnp.pad(V0[:3], ((0, 13), (0, 0)))
    ea0, sde0, e1 = _tc_call(
        _edge_fwd0_body, E, EB,
        [gs0, gd0],
        [V0p, V0[3].reshape(1, -1), c0.reshape(1, -1), V1, c1.reshape(1, -1),
         C[0], be0[0], We1[0], be1[0]],
        [DH, DH, DH], "edge_fwd0")

    agg0 = _segsum(ea0, d_lo, d_hi, N)
    x1, sdn0, xs1, xd1 = _tc_call(
        functools.partial(_node_fwd_body, True), N, NB,
        [xenc, agg0],
        [Wa[0], Wb[0], bn0[0], Wn1[0], bn1[0], A[1], B[1]],
        [DH, DH, DH, DH], "node_fwd0")

    gxs1, gxd1 = _sc_gather2(xs1, xd1, src1, dest1, DH)
    ea1, sde1 = _tc_call(
        _edge_fwd1_body, E, EB,
        [e1, gxs1, gxd1],
        [C[1], be0[1], We1[1], be1[1]],
        [DH, DH], "edge_fwd1")

    agg1 = _segsum(ea1, d_lo, d_hi, N)
    x2, sdn1 = _tc_call(
        functools.partial(_node_fwd_body, False), N, NB,
        [x1, agg1],
        [Wa[1], Wb[1], bn0[1], Wn1[1], bn1[1]],
        [DH, DH], "node_fwd1")

    (WE0, bE0), (WE1, bE1) = params["dec_E"]
    (WS0, bS0), (WS1, bS1) = params["dec_S"]
    (WL0, bL0), (WL1, bL1) = params["dec_L"]
    (WM0, bM0), (WM1, bM1) = params["dec_M"]

    import numpy as _np
    rL, cL = _np.tril_indices(6, -1)
    PL = _np.zeros((15, 36), _np.float32)
    for j, (a, b) in enumerate(zip(rL, cL)):
        PL[j, a * 6 + b] = 1.0
        PL[j, b * 6 + a] = -1.0
    PLj = jnp.asarray(PL)
    WL1f = WL1 @ PLj
    bL1f = (bL1 @ PLj).reshape(1, -1)

    es8, gxE, gxS, lcols, mmt = _tc_call(
        _dec_body, N, NB,
        [x2],
        [WE0, bE0.reshape(1, -1), WE1[:, 0].reshape(1, -1), bE1.reshape(1, -1),
         WS0, bS0.reshape(1, -1), WS1[:, 0].reshape(1, -1), bS1.reshape(1, -1),
         WL0, bL0.reshape(1, -1), WL1f, bL1f,
         WM0, bM0.reshape(1, -1), WM1, bM1.reshape(1, -1),
         WE0.T, WS0.T],
        [8, DH, DH, 36, 36], "dec")

    gaE1, gaS1, gpE1, gpS1 = _tc_call(
        _node_bwd1_body, N, NB,
        [gxE, gxS, sdn1],
        [Wn1[1].T, Wa[1].T, Wb[1].T],
        [DH, DH, DH, DH], "node_bwd1")

    ggE1, ggS1 = _sc_gather2(gaE1, gaS1, dest1, dest1, DH)
    gprE1, gprS1, geE, geS = _tc_call(
        _edge_bwd1_body, E, EB,
        [ggE1, ggS1, sde1],
        [We1[1].T, C[1].T],
        [DH, DH, DH, DH], "edge_bwd1")

    ssE1 = _segsum(gprE1, s_lo, s_hi, N)
    sdE1 = _segsum(gprE1, d_lo, d_hi, N)
    ssS1 = _segsum(gprS1, s_lo, s_hi, N)
    sdS1 = _segsum(gprS1, d_lo, d_hi, N)

    gaE0, gaS0, gpE0, gpS0 = _tc_call(
        _node_bwd_mid_body, N, NB,
        [gpE1, gpS1, ssE1, sdE1, ssS1, sdS1, sdn0],
        [A[1].T, B[1].T, Wn1[0].T, Wa[0].T, Wb[0].T],
        [DH, DH, DH, DH], "node_bwd_mid")

    ggE0, ggS0 = _sc_gather2(gaE0, gaS0, dest1, dest1, DH)
    gprE0, gprS0 = _tc_call(
        _edge_bwd0_body, E, EB,
        [ggE0, ggS0, geE, geS, sde0],
        [We1[0].T],
        [DH, DH], "edge_bwd0")

    ssE0 = _segsum(gprE0, s_lo, s_hi, N)
    sdE0 = _segsum(gprE0, d_lo, d_hi, N)
    ssS0 = _segsum(gprS0, s_lo, s_hi, N)
    sdS0 = _segsum(gprS0, d_lo, d_hi, N)

    dzE8, dzS8 = _tc_call(
        _node_bwd_final_body, N, NB,
        [gpE0, gpS0, ssE0, sdE0, ssS0, sdS0, sdenc],
        [A[0].T, B[0].T, W1e.T, W0e.T],
        [8, 8], "node_bwd_final")

    L = lcols.reshape(N, 6, 6)
    M = mmt.reshape(N, 6, 6)
    dEdz = dzE8[:, :6].reshape(N, 6, 1)
    dSdz = dzS8[:, :6].reshape(N, 6, 1)
    E_out = es8[:, 0:1]
    S_out = es8[:, 1:2]
    return (L, M, dEdz, dSdz, E_out, S_out)


# speedup vs baseline: 1.2968x; 1.2483x over previous
import functools

import jax
import jax.numpy as jnp
from jax import lax
from jax.experimental import pallas as pl
from jax.experimental.pallas import tpu as pltpu
from jax.experimental.pallas import tpu_sc as plsc

N_NODES = 10000
DH = 128
NB = 1000
EB = 2000
NC = 2
NS = 16
IDXW = 80
HN = 5120
HNP = 5248
F32 = jnp.float32


def _silu_and_d(pre):
    s = jax.nn.sigmoid(pre)
    return pre * s, s * (1.0 + pre * (1.0 - s))


def _row_spec(arr_like, blk):
    shp = arr_like.shape
    if len(shp) == 2:
        return pl.BlockSpec((blk, shp[1]), lambda i: (i, 0))
    return pl.BlockSpec((shp[0], blk, shp[2]), lambda i: (0, i, 0))


def _full_spec(arr_like):
    shp = arr_like.shape
    return pl.BlockSpec(shp, lambda i, _r=len(shp): (0,) * _r)


def _tc_call(body, n_rows, blk, row_in, full_in, out_rowdims, name):
    in_specs = [_row_spec(a, blk) for a in row_in] + [_full_spec(a) for a in full_in]
    out_specs = [pl.BlockSpec((blk, d), lambda i: (i, 0)) for d in out_rowdims]
    out_shape = [jax.ShapeDtypeStruct((n_rows, d), F32) for d in out_rowdims]
    return pl.pallas_call(
        body,
        grid=(n_rows // blk,),
        in_specs=in_specs,
        out_specs=out_specs,
        out_shape=out_shape,
        name=name,
    )(*row_in, *full_in)


BF16 = jnp.bfloat16


def _mm(a, b):
    return jnp.dot(a.astype(BF16), b.astype(BF16), preferred_element_type=F32)


def _r16(a):
    return a.astype(BF16).astype(F32)


def _sc_gather2(table0, table1, idx0, idx1, D):
    E = idx0.shape[0]
    w = IDXW if D <= 128 else IDXW // 2
    nj = 5
    ce = nj * w
    pairs = E // NS // (2 * ce)
    per_e = E // NS
    mesh = plsc.VectorSubcoreMesh(core_axis_name="c", subcore_axis_name="s")

    @functools.partial(
        pl.kernel,
        out_type=(
            jax.ShapeDtypeStruct((E, D), F32),
            jax.ShapeDtypeStruct((E, D), F32),
        ),
        mesh=mesh,
        scratch_types=[
            [[pltpu.VMEM((w,), jnp.int32) for _ in range(nj)] for _ in range(2)],
            [pltpu.VMEM((ce, D), F32) for _ in range(2)],
            pltpu.SemaphoreType.DMA,
            pltpu.SemaphoreType.DMA,
            pltpu.SemaphoreType.DMA,
        ],
    )
    def k(t0, t1, i0, i1, o0, o1, idx_b, rows_b, sem_i, sem_g, sem_o):
        cid = lax.axis_index("c")
        sid = lax.axis_index("s")

        def stream(t, idx, out):
            def pair_body(p, carry):
                b0 = sid * per_e + 2 * p * ce
                b1 = b0 + ce
                i0s = [pltpu.async_copy(idx.at[pl.ds(b0 + j * w, w)],
                                        idx_b[0][j], sem_i) for j in range(nj)]
                i1s = [pltpu.async_copy(idx.at[pl.ds(b1 + j * w, w)],
                                        idx_b[1][j], sem_i) for j in range(nj)]
                for cp in i0s:
                    cp.wait()
                g0s = [pltpu.async_copy(t.at[idx_b[0][j]],
                                        rows_b[0].at[pl.ds(j * w, w)], sem_g)
                       for j in range(nj)]
                for cp in i1s:
                    cp.wait()
                for cp in g0s:
                    cp.wait()
                o0c = pltpu.async_copy(rows_b[0], out.at[pl.ds(b0, ce)], sem_o)
                g1s = [pltpu.async_copy(t.at[idx_b[1][j]],
                                        rows_b[1].at[pl.ds(j * w, w)], sem_g)
                       for j in range(nj)]
                for cp in g1s:
                    cp.wait()
                o0c.wait()
                pltpu.sync_copy(rows_b[1], out.at[pl.ds(b1, ce)])
                return carry

            lax.fori_loop(0, pairs, pair_body, 0)

        @pl.when(cid == 0)
        def _():
            stream(t0, i0, o0)

        @pl.when(cid != 0)
        def _():
            stream(t1, i1, o1)

    return k(table0, table1, idx0, idx1)


def _sc_scatter_rs(vals, idx_lo, idx_hi):
    E = idx_lo.shape[0]
    w = IDXW // 2
    nj = 5
    ce = nj * w
    pairs = E // NS // (2 * ce)
    per_e = E // NS
    per_sub = HNP // NS
    mesh = plsc.VectorSubcoreMesh(core_axis_name="c", subcore_axis_name="s")
    zeros = jnp.zeros((per_sub, DH), F32)

    @functools.partial(
        pl.kernel,
        out_type=jax.ShapeDtypeStruct((NC, HNP, DH), F32),
        mesh=mesh,
        scratch_types=[
            [[pltpu.VMEM((w,), jnp.int32) for _ in range(nj)] for _ in range(2)],
            [pltpu.VMEM((ce, DH), F32) for _ in range(2)],
            pltpu.VMEM_SHARED((HNP, DH), F32),
            pltpu.SemaphoreType.DMA,
            pltpu.SemaphoreType.DMA,
            pltpu.SemaphoreType.DMA,
        ],
    )
    def k(v, z, ilo, ihi, out, idx_b, rows_b, acc, sem_i, sem_v, sem_s):
        cid = lax.axis_index("c")
        sid = lax.axis_index("s")
        ob = sid * per_sub
        pltpu.sync_copy(z, acc.at[pl.ds(ob, per_sub)])
        plsc.subcore_barrier()

        def stream(idx):
            def pair_body(p, carry):
                b0 = sid * per_e + 2 * p * ce
                b1 = b0 + ce
                v0c = pltpu.async_copy(v.at[pl.ds(b0, ce)], rows_b[0], sem_v)
                i0s = [pltpu.async_copy(idx.at[pl.ds(b0 + j * w, w)],
                                        idx_b[0][j], sem_i) for j in range(nj)]
                v1c = pltpu.async_copy(v.at[pl.ds(b1, ce)], rows_b[1], sem_v)
                i1s = [pltpu.async_copy(idx.at[pl.ds(b1 + j * w, w)],
                                        idx_b[1][j], sem_i) for j in range(nj)]
                v0c.wait()
                for cp in i0s:
                    cp.wait()
                a0s = [pltpu.async_copy(rows_b[0].at[pl.ds(j * w, w)],
                                        acc.at[idx_b[0][j]], sem_s, add=True)
                       for j in range(nj)]
                v1c.wait()
                for cp in i1s:
                    cp.wait()
                for cp in a0s:
                    cp.wait()
                a1s = [pltpu.async_copy(rows_b[1].at[pl.ds(j * w, w)],
                                        acc.at[idx_b[1][j]], sem_s, add=True)
                       for j in range(nj)]
                for cp in a1s:
                    cp.wait()
                return carry

            lax.fori_loop(0, pairs, pair_body, 0)

        @pl.when(cid == 0)
        def _():
            stream(ilo)

        @pl.when(cid != 0)
        def _():
            stream(ihi)

        plsc.subcore_barrier()
        pltpu.sync_copy(acc.at[pl.ds(ob, per_sub)],
                        out.at[cid].at[pl.ds(ob, per_sub)])

    return k(vals, zeros, idx_lo, idx_hi)


def _segsum(vals, idx_lo, idx_hi, n):
    out = _sc_scatter_rs(vals, idx_lo, idx_hi)
    return jnp.concatenate([out[0, :HN], out[1, :n - HN]], axis=0)


NPAD = 10240


def _sc_scatter_two(vals, idxA, idxB):
    E = idxA.shape[0]
    w = IDXW
    ce = w
    pairs = E // NS // (2 * ce)
    per_e = E // NS
    per_sub = NPAD // NS
    mesh = plsc.VectorSubcoreMesh(core_axis_name="c", subcore_axis_name="s")
    zeros = jnp.zeros((per_sub, DH), F32)

    @functools.partial(
        pl.kernel,
        out_type=jax.ShapeDtypeStruct((NC, NPAD, DH), F32),
        mesh=mesh,
        scratch_types=[
            [pltpu.VMEM((w,), jnp.int32) for _ in range(2)],
            [pltpu.VMEM((ce, DH), F32) for _ in range(2)],
            pltpu.VMEM_SHARED((NPAD, DH), F32),
            pltpu.SemaphoreType.DMA,
            pltpu.SemaphoreType.DMA,
            pltpu.SemaphoreType.DMA,
        ],
    )
    def k(v, z, iA, iB, out, idx_b, rows_b, acc, sem_i, sem_v, sem_s):
        cid = lax.axis_index("c")
        sid = lax.axis_index("s")
        ob = sid * per_sub
        pltpu.sync_copy(z, acc.at[pl.ds(ob, per_sub)])
        plsc.subcore_barrier()

        def stream(idx):
            def pair_body(p, carry):
                b0 = sid * per_e + 2 * p * ce
                b1 = b0 + ce
                v0c = pltpu.async_copy(v.at[pl.ds(b0, ce)], rows_b[0], sem_v)
                i0c = pltpu.async_copy(idx.at[pl.ds(b0, w)], idx_b[0], sem_i)
                v1c = pltpu.async_copy(v.at[pl.ds(b1, ce)], rows_b[1], sem_v)
                i1c = pltpu.async_copy(idx.at[pl.ds(b1, w)], idx_b[1], sem_i)
                v0c.wait()
                i0c.wait()
                a0 = pltpu.async_copy(rows_b[0], acc.at[idx_b[0]], sem_s,
                                      add=True)
                v1c.wait()
                i1c.wait()
                a0.wait()
                a1 = pltpu.async_copy(rows_b[1], acc.at[idx_b[1]], sem_s,
                                      add=True)
                a1.wait()
                return carry

            lax.fori_loop(0, pairs, pair_body, 0)

        @pl.when(cid == 0)
        def _():
            stream(iA)

        @pl.when(cid != 0)
        def _():
            stream(iB)

        plsc.subcore_barrier()
        pltpu.sync_copy(acc.at[pl.ds(ob, per_sub)],
                        out.at[cid].at[pl.ds(ob, per_sub)])

    return k(vals, zeros, idxA, idxB)


def _enc_node_body(z, n, q16, W0z, W0n, b0, W1, b1, A0, B0,
                   xenc_o, sd_o, ts_o, td_o):
    pre = _mm(z[...], W0z[...]) + _mm(n[...], W0n[...]) + b0[...]
    h, sd = _silu_and_d(pre)
    sd_o[...] = sd
    xe = _mm(h, W1[...]) + b1[...]
    xenc_o[...] = xe
    qv = q16[...]
    pad = jnp.zeros((qv.shape[0], 112), F32)
    ts_o[...] = jnp.concatenate([_mm(xe, A0[...]), qv, pad], axis=1)
    td_o[...] = jnp.concatenate([_mm(xe, B0[...]), qv, pad], axis=1)


def _edge_fwd0_body(gs, gd, V0p, v0n, c0, V1, c1, C0, be0, W1, be1,
                    ea_o, sd_o, e1_o):
    d16 = gs[:, DH:DH + 16] - gd[:, DH:DH + 16]
    nrm = jnp.sqrt(jnp.sum(d16 * d16, axis=1, keepdims=True))
    pre0 = _mm(d16, V0p[...]) + _r16(nrm) * _r16(v0n[...]) + c0[...]
    h0, _ = _silu_and_d(pre0)
    e0 = _mm(h0, V1[...]) + c1[...]
    pre = _mm(e0, C0[...]) + gs[:, :DH] + gd[:, :DH] + be0[...]
    h, sd = _silu_and_d(pre)
    sd_o[...] = sd
    ea = _mm(h, W1[...]) + be1[...]
    ea_o[...] = ea
    e1_o[...] = e0 + ea


def _edge_fwd1_body(e, gs, gd, C, be0, W1, be1, ea_o, sd_o):
    pre = _mm(e[...], C[...]) + gs[...] + gd[...] + be0[...]
    h, sd = _silu_and_d(pre)
    sd_o[...] = sd
    ea_o[...] = _mm(h, W1[...]) + be1[...]


def _node_fwd_body(with_tables, x, agg, Wa, Wb, bn0, W1, bn1, *rest):
    if with_tables:
        A, B, xn_o, sd_o, xs_o, xd_o = rest
    else:
        xn_o, sd_o = rest
    pre = _mm(x[...], Wa[...]) + _mm(agg[...], Wb[...]) + bn0[...]
    h, sd = _silu_and_d(pre)
    sd_o[...] = sd
    xn = x[...] + _mm(h, W1[...]) + bn1[...]
    xn_o[...] = xn
    if with_tables:
        xs_o[...] = _mm(xn, A[...])
        xd_o[...] = _mm(xn, B[...])


_MMT_TERMS = None


def _mmt_terms():
    global _MMT_TERMS
    if _MMT_TERMS is None:
        pos = {}
        j = 0
        for a in range(6):
            for b in range(a + 1):
                pos[(a, b)] = j
                j += 1
        terms = []
        for a in range(6):
            for b in range(6):
                t = [(pos[(a, c)], pos[(b, c)]) for c in range(min(a, b) + 1)]
                terms.append(t)
        _MMT_TERMS = terms
    return _MMT_TERMS


def _dec_body(x, WE0, bE0, wE1, bE1, WS0, bS0, wS1, bS1,
              WL0, bL0, WL1f, bL1f, WM0, bM0, WM1, bM1,
              WE0T, WS0T,
              es_o, gxE_o, gxS_o, lcols_o, mmt_o):
    xv = x[...]
    preE = _mm(xv, WE0[...]) + bE0[...]
    hE, sdE = _silu_and_d(preE)
    Ev = jnp.sum(_r16(hE) * _r16(wE1[...]), axis=1, keepdims=True) + bE1[...]
    gxE_o[...] = _mm(sdE * _r16(wE1[...]), WE0T[...])
    preS = _mm(xv, WS0[...]) + bS0[...]
    hS, sdS = _silu_and_d(preS)
    Sv = jnp.sum(_r16(hS) * _r16(wS1[...]), axis=1, keepdims=True) + bS1[...]
    gxS_o[...] = _mm(sdS * _r16(wS1[...]), WS0T[...])
    z1 = jnp.zeros_like(Ev)
    es_o[...] = jnp.concatenate([Ev, Sv, z1, z1, z1, z1, z1, z1], axis=1)
    preL = _mm(xv, WL0[...]) + bL0[...]
    hL, _ = _silu_and_d(preL)
    lcols_o[...] = _mm(hL, WL1f[...]) + bL1f[...]
    preM = _mm(xv, WM0[...]) + bM0[...]
    hM, _ = _silu_and_d(preM)
    m = _r16(_mm(hM, WM1[...]) + bM1[...])
    cols = []
    for t in _mmt_terms():
        acc = m[:, t[0][0]:t[0][0] + 1] * m[:, t[0][1]:t[0][1] + 1]
        for (pa, pb) in t[1:]:
            acc = acc + m[:, pa:pa + 1] * m[:, pb:pb + 1]
        cols.append(acc)
    mmt_o[...] = jnp.concatenate(cols, axis=1)


def _node_bwd1_body(gE, gS, sdn, Wn1T, WaT, WbT,
                    gaE_o, gaS_o, gpE_o, gpS_o):
    for g, ga_o, gp_o in ((gE, gaE_o, gpE_o), (gS, gaS_o, gpS_o)):
        gpren = _mm(g[...], Wn1T[...]) * sdn[...]
        ga_o[...] = _mm(gpren, WbT[...])
        gp_o[...] = g[...] + _mm(gpren, WaT[...])


def _edge_bwd1_body(ggE, ggS, sd, We1T, CT,
                    gprE_o, gprS_o, geE_o, geS_o):
    for gg, gpr_o, ge_o in ((ggE, gprE_o, geE_o), (ggS, gprS_o, geS_o)):
        gpr = _mm(gg[...], We1T[...]) * sd[...]
        gpr_o[...] = gpr
        ge_o[...] = _mm(gpr, CT[...])


def _edge_bwd0_body(ggE, ggS, geE, geS, sd, We1T, gprE_o, gprS_o):
    for gg, ge, gpr_o in ((ggE, geE, gprE_o), (ggS, geS, gprS_o)):
        gpr_o[...] = _mm(gg[...] + ge[...], We1T[...]) * sd[...]


def _node_bwd_mid_body(gpE, gpS, ssE, sdE, ssS, sdS, sdn0,
                       A1T, B1T, Wn1T0, WaT0, WbT0,
                       gaE_o, gaS_o, gpE_o, gpS_o):
    for gp, ss, sdst, ga_o, gp_o in (
            (gpE, ssE, sdE, gaE_o, gpE_o), (gpS, ssS, sdS, gaS_o, gpS_o)):
        g1 = gp[...] + _mm(ss[...], A1T[...]) + _mm(sdst[...], B1T[...])
        gpren = _mm(g1, Wn1T0[...]) * sdn0[...]
        ga_o[...] = _mm(gpren, WbT0[...])
        gp_o[...] = g1 + _mm(gpren, WaT0[...])


def _node_bwd_final_body(gpE, gpS, ssE, sdE, ssS, sdS, sdenc,
                         A0T, B0T, W1eT, W0eT, dzE_o, dzS_o):
    for gp, ss, sdst, dz_o in (
            (gpE, ssE, sdE, dzE_o), (gpS, ssS, sdS, dzS_o)):
        g0 = gp[...] + _mm(ss[...], A0T[...]) + _mm(sdst[...], B0T[...])
        gpre = _mm(g0, W1eT[...]) * sdenc[...]
        dz_o[...] = _mm(gpre, W0eT[...])


def kernel(z, n, edge_index, q_0, params):
    N = z.shape[0]
    E = edge_index.shape[1]
    src1 = edge_index[0]
    dest1 = edge_index[1]
    s_lo = jnp.where(src1 < HN, src1, HN)
    s_hi = jnp.where(src1 >= HN, src1 - HN, HN)
    d_lo = jnp.where(dest1 < HN, dest1, HN)
    d_hi = jnp.where(dest1 >= HN, dest1 - HN, HN)

    (W0e, b0e), (W1e, b1e) = params["enc_node"]
    (V0, c0), (V1, c1) = params["enc_edge"]
    em = params["edge_mlps"]
    nm = params["node_mlps"]
    C = [em[p][0][0][:DH] for p in range(2)]
    A = [em[p][0][0][DH:2 * DH] for p in range(2)]
    B = [em[p][0][0][2 * DH:] for p in range(2)]
    be0 = [em[p][0][1].reshape(1, -1) for p in range(2)]
    We1 = [em[p][1][0] for p in range(2)]
    be1 = [em[p][1][1].reshape(1, -1) for p in range(2)]
    Wa = [nm[p][0][0][:DH] for p in range(2)]
    Wb = [nm[p][0][0][DH:] for p in range(2)]
    bn0 = [nm[p][0][1].reshape(1, -1) for p in range(2)]
    Wn1 = [nm[p][1][0] for p in range(2)]
    bn1 = [nm[p][1][1].reshape(1, -1) for p in range(2)]

    q16 = jnp.pad(q_0, ((0, 0), (0, 13)))
    xenc, sdenc, tbl_s, tbl_d = _tc_call(
        _enc_node_body, N, NB,
        [z, n, q16],
        [W0e[:6], W0e[6:], b0e.reshape(1, -1), W1e, b1e.reshape(1, -1),
         A[0], B[0]],
        [DH, DH, 256, 256], "enc_node")

    gs0, gd0 = _sc_gather2(tbl_s, tbl_d, src1, dest1, 256)

    V0p = jnp.pad(V0[:3], ((0, 13), (0, 0)))
    ea0, sde0, e1 = _tc_call(
        _edge_fwd0_body, E, EB,
        [gs0, gd0],
        [V0p, V0[3].reshape(1, -1), c0.reshape(1, -1), V1, c1.reshape(1, -1),
         C[0], be0[0], We1[0], be1[0]],
        [DH, DH, DH], "edge_fwd0")

    agg0 = _segsum(ea0, d_lo, d_hi, N)
    x1, sdn0, xs1, xd1 = _tc_call(
        functools.partial(_node_fwd_body, True), N, NB,
        [xenc, agg0],
        [Wa[0], Wb[0], bn0[0], Wn1[0], bn1[0], A[1], B[1]],
        [DH, DH, DH, DH], "node_fwd0")

    gxs1, gxd1 = _sc_gather2(xs1, xd1, src1, dest1, DH)
    ea1, sde1 = _tc_call(
        _edge_fwd1_body, E, EB,
        [e1, gxs1, gxd1],
        [C[1], be0[1], We1[1], be1[1]],
        [DH, DH], "edge_fwd1")

    agg1 = _segsum(ea1, d_lo, d_hi, N)
    x2, sdn1 = _tc_call(
        functools.partial(_node_fwd_body, False), N, NB,
        [x1, agg1],
        [Wa[1], Wb[1], bn0[1], Wn1[1], bn1[1]],
        [DH, DH], "node_fwd1")

    (WE0, bE0), (WE1, bE1) = params["dec_E"]
    (WS0, bS0), (WS1, bS1) = params["dec_S"]
    (WL0, bL0), (WL1, bL1) = params["dec_L"]
    (WM0, bM0), (WM1, bM1) = params["dec_M"]

    import numpy as _np
    rL, cL = _np.tril_indices(6, -1)
    PL = _np.zeros((15, 36), _np.float32)
    for j, (a, b) in enumerate(zip(rL, cL)):
        PL[j, a * 6 + b] = 1.0
        PL[j, b * 6 + a] = -1.0
    PLj = jnp.asarray(PL)
    WL1f = WL1 @ PLj
    bL1f = (bL1 @ PLj).reshape(1, -1)

    es8, gxE, gxS, lcols, mmt = _tc_call(
        _dec_body, N, NB,
        [x2],
        [WE0, bE0.reshape(1, -1), WE1[:, 0].reshape(1, -1), bE1.reshape(1, -1),
         WS0, bS0.reshape(1, -1), WS1[:, 0].reshape(1, -1), bS1.reshape(1, -1),
         WL0, bL0.reshape(1, -1), WL1f, bL1f,
         WM0, bM0.reshape(1, -1), WM1, bM1.reshape(1, -1),
         WE0.T, WS0.T],
        [8, DH, DH, 36, 36], "dec")

    gaE1, gaS1, gpE1, gpS1 = _tc_call(
        _node_bwd1_body, N, NB,
        [gxE, gxS, sdn1],
        [Wn1[1].T, Wa[1].T, Wb[1].T],
        [DH, DH, DH, DH], "node_bwd1")

    ggE1, ggS1 = _sc_gather2(gaE1, gaS1, dest1, dest1, DH)
    gprE1, gprS1, geE, geS = _tc_call(
        _edge_bwd1_body, E, EB,
        [ggE1, ggS1, sde1],
        [We1[1].T, C[1].T],
        [DH, DH, DH, DH], "edge_bwd1")

    sE1 = _sc_scatter_two(gprE1, src1, dest1)
    sS1 = _sc_scatter_two(gprS1, src1, dest1)
    ssE1, sdE1 = sE1[0, :N], sE1[1, :N]
    ssS1, sdS1 = sS1[0, :N], sS1[1, :N]

    gaE0, gaS0, gpE0, gpS0 = _tc_call(
        _node_bwd_mid_body, N, NB,
        [gpE1, gpS1, ssE1, sdE1, ssS1, sdS1, sdn0],
        [A[1].T, B[1].T, Wn1[0].T, Wa[0].T, Wb[0].T],
        [DH, DH, DH, DH], "node_bwd_mid")

    ggE0, ggS0 = _sc_gather2(gaE0, gaS0, dest1, dest1, DH)
    gprE0, gprS0 = _tc_call(
        _edge_bwd0_body, E, EB,
        [ggE0, ggS0, geE, geS, sde0],
        [We1[0].T],
        [DH, DH], "edge_bwd0")

    sE0 = _sc_scatter_two(gprE0, src1, dest1)
    sS0 = _sc_scatter_two(gprS0, src1, dest1)
    ssE0, sdE0 = sE0[0, :N], sE0[1, :N]
    ssS0, sdS0 = sS0[0, :N], sS0[1, :N]

    dzE8, dzS8 = _tc_call(
        _node_bwd_final_body, N, NB,
        [gpE0, gpS0, ssE0, sdE0, ssS0, sdS0, sdenc],
        [A[0].T, B[0].T, W1e.T, W0e.T],
        [8, 8], "node_bwd_final")

    L = lcols.reshape(N, 6, 6)
    M = mmt.reshape(N, 6, 6)
    dEdz = dzE8[:, :6].reshape(N, 6, 1)
    dSdz = dzS8[:, :6].reshape(N, 6, 1)
    E_out = es8[:, 0:1]
    S_out = es8[:, 1:2]
    return (L, M, dEdz, dSdz, E_out, S_out)


# speedup vs baseline: 1.2998x; 1.0023x over previous
import functools

import jax
import jax.numpy as jnp
from jax import lax
from jax.experimental import pallas as pl
from jax.experimental.pallas import tpu as pltpu
from jax.experimental.pallas import tpu_sc as plsc

N_NODES = 10000
DH = 128
NB = 1000
EB = 2000
NC = 2
NS = 16
IDXW = 80
HN = 5120
HNP = 5248
F32 = jnp.float32


def _silu_and_d(pre):
    s = jax.nn.sigmoid(pre)
    return pre * s, s * (1.0 + pre * (1.0 - s))


def _row_spec(arr_like, blk):
    shp = arr_like.shape
    if len(shp) == 2:
        return pl.BlockSpec((blk, shp[1]), lambda i: (i, 0))
    return pl.BlockSpec((shp[0], blk, shp[2]), lambda i: (0, i, 0))


def _full_spec(arr_like):
    shp = arr_like.shape
    return pl.BlockSpec(shp, lambda i, _r=len(shp): (0,) * _r)


def _tc_call(body, n_rows, blk, row_in, full_in, out_rowdims, name):
    in_specs = [_row_spec(a, blk) for a in row_in] + [_full_spec(a) for a in full_in]
    out_specs = [pl.BlockSpec((blk, d), lambda i: (i, 0)) for d in out_rowdims]
    out_shape = [jax.ShapeDtypeStruct((n_rows, d), F32) for d in out_rowdims]
    return pl.pallas_call(
        body,
        grid=(n_rows // blk,),
        in_specs=in_specs,
        out_specs=out_specs,
        out_shape=out_shape,
        name=name,
    )(*row_in, *full_in)


BF16 = jnp.bfloat16


def _mm(a, b):
    return jnp.dot(a.astype(BF16), b.astype(BF16), preferred_element_type=F32)


def _r16(a):
    return a.astype(BF16).astype(F32)


def _sc_gather2(table0, table1, idx0, idx1, D):
    E = idx0.shape[0]
    w = IDXW if D <= 128 else IDXW // 2
    nj = 5
    ce = nj * w
    pairs = E // NS // (2 * ce)
    per_e = E // NS
    mesh = plsc.VectorSubcoreMesh(core_axis_name="c", subcore_axis_name="s")

    @functools.partial(
        pl.kernel,
        out_type=(
            jax.ShapeDtypeStruct((E, D), F32),
            jax.ShapeDtypeStruct((E, D), F32),
        ),
        mesh=mesh,
        scratch_types=[
            [[pltpu.VMEM((w,), jnp.int32) for _ in range(nj)] for _ in range(2)],
            [pltpu.VMEM((ce, D), F32) for _ in range(2)],
            pltpu.SemaphoreType.DMA,
            pltpu.SemaphoreType.DMA,
            pltpu.SemaphoreType.DMA,
        ],
    )
    def k(t0, t1, i0, i1, o0, o1, idx_b, rows_b, sem_i, sem_g, sem_o):
        cid = lax.axis_index("c")
        sid = lax.axis_index("s")

        def stream(t, idx, out):
            def pair_body(p, carry):
                b0 = sid * per_e + 2 * p * ce
                b1 = b0 + ce
                i0s = [pltpu.async_copy(idx.at[pl.ds(b0 + j * w, w)],
                                        idx_b[0][j], sem_i) for j in range(nj)]
                i1s = [pltpu.async_copy(idx.at[pl.ds(b1 + j * w, w)],
                                        idx_b[1][j], sem_i) for j in range(nj)]
                for cp in i0s:
                    cp.wait()
                g0s = [pltpu.async_copy(t.at[idx_b[0][j]],
                                        rows_b[0].at[pl.ds(j * w, w)], sem_g)
                       for j in range(nj)]
                for cp in i1s:
                    cp.wait()
                for cp in g0s:
                    cp.wait()
                o0c = pltpu.async_copy(rows_b[0], out.at[pl.ds(b0, ce)], sem_o)
                g1s = [pltpu.async_copy(t.at[idx_b[1][j]],
                                        rows_b[1].at[pl.ds(j * w, w)], sem_g)
                       for j in range(nj)]
                for cp in g1s:
                    cp.wait()
                o0c.wait()
                pltpu.sync_copy(rows_b[1], out.at[pl.ds(b1, ce)])
                return carry

            lax.fori_loop(0, pairs, pair_body, 0)

        @pl.when(cid == 0)
        def _():
            stream(t0, i0, o0)

        @pl.when(cid != 0)
        def _():
            stream(t1, i1, o1)

    return k(table0, table1, idx0, idx1)


def _sc_gather2_staged(table0, table1, idx0, idx1):
    E = idx0.shape[0]
    w = IDXW
    ce = w
    pairs = E // NS // (2 * ce)
    per_e = E // NS
    per_tab = NPAD // NS
    mesh = plsc.VectorSubcoreMesh(core_axis_name="c", subcore_axis_name="s")

    @functools.partial(
        pl.kernel,
        out_type=(
            jax.ShapeDtypeStruct((E, DH), F32),
            jax.ShapeDtypeStruct((E, DH), F32),
        ),
        mesh=mesh,
        scratch_types=[
            [pltpu.VMEM((w,), jnp.int32) for _ in range(2)],
            [pltpu.VMEM((ce, DH), F32) for _ in range(2)],
            pltpu.VMEM_SHARED((NPAD, DH), F32),
            pltpu.SemaphoreType.DMA,
            pltpu.SemaphoreType.DMA,
            pltpu.SemaphoreType.DMA,
        ],
    )
    def k(t0, t1, i0, i1, o0, o1, idx_b, rows_b, acc, sem_i, sem_g, sem_o):
        cid = lax.axis_index("c")
        sid = lax.axis_index("s")
        tb = sid * per_tab

        def stream(t, idx, out):
            pltpu.sync_copy(t.at[pl.ds(tb, per_tab)], acc.at[pl.ds(tb, per_tab)])
            plsc.subcore_barrier()

            def pair_body(p, carry):
                b0 = sid * per_e + 2 * p * ce
                b1 = b0 + ce
                i0c = pltpu.async_copy(idx.at[pl.ds(b0, w)], idx_b[0], sem_i)
                i1c = pltpu.async_copy(idx.at[pl.ds(b1, w)], idx_b[1], sem_i)
                i0c.wait()
                g0 = pltpu.async_copy(acc.at[idx_b[0]], rows_b[0], sem_g)
                i1c.wait()
                g0.wait()
                o0c = pltpu.async_copy(rows_b[0], out.at[pl.ds(b0, ce)], sem_o)
                g1 = pltpu.async_copy(acc.at[idx_b[1]], rows_b[1], sem_g)
                g1.wait()
                o0c.wait()
                pltpu.sync_copy(rows_b[1], out.at[pl.ds(b1, ce)])
                return carry

            lax.fori_loop(0, pairs, pair_body, 0)

        @pl.when(cid == 0)
        def _():
            stream(t0, i0, o0)

        @pl.when(cid != 0)
        def _():
            stream(t1, i1, o1)

    pad0 = jnp.pad(table0, ((0, NPAD - table0.shape[0]), (0, 0)))
    pad1 = jnp.pad(table1, ((0, NPAD - table1.shape[0]), (0, 0)))
    return k(pad0, pad1, idx0, idx1)


def _sc_scatter_rs(vals, idx_lo, idx_hi):
    E = idx_lo.shape[0]
    w = IDXW // 2
    nj = 5
    ce = nj * w
    pairs = E // NS // (2 * ce)
    per_e = E // NS
    per_sub = HNP // NS
    mesh = plsc.VectorSubcoreMesh(core_axis_name="c", subcore_axis_name="s")
    zeros = jnp.zeros((per_sub, DH), F32)

    @functools.partial(
        pl.kernel,
        out_type=jax.ShapeDtypeStruct((NC, HNP, DH), F32),
        mesh=mesh,
        scratch_types=[
            [[pltpu.VMEM((w,), jnp.int32) for _ in range(nj)] for _ in range(2)],
            [pltpu.VMEM((ce, DH), F32) for _ in range(2)],
            pltpu.VMEM_SHARED((HNP, DH), F32),
            pltpu.SemaphoreType.DMA,
            pltpu.SemaphoreType.DMA,
            pltpu.SemaphoreType.DMA,
        ],
    )
    def k(v, z, ilo, ihi, out, idx_b, rows_b, acc, sem_i, sem_v, sem_s):
        cid = lax.axis_index("c")
        sid = lax.axis_index("s")
        ob = sid * per_sub
        pltpu.sync_copy(z, acc.at[pl.ds(ob, per_sub)])
        plsc.subcore_barrier()

        def stream(idx):
            def pair_body(p, carry):
                b0 = sid * per_e + 2 * p * ce
                b1 = b0 + ce
                v0c = pltpu.async_copy(v.at[pl.ds(b0, ce)], rows_b[0], sem_v)
                i0s = [pltpu.async_copy(idx.at[pl.ds(b0 + j * w, w)],
                                        idx_b[0][j], sem_i) for j in range(nj)]
                v1c = pltpu.async_copy(v.at[pl.ds(b1, ce)], rows_b[1], sem_v)
                i1s = [pltpu.async_copy(idx.at[pl.ds(b1 + j * w, w)],
                                        idx_b[1][j], sem_i) for j in range(nj)]
                v0c.wait()
                for cp in i0s:
                    cp.wait()
                a0s = [pltpu.async_copy(rows_b[0].at[pl.ds(j * w, w)],
                                        acc.at[idx_b[0][j]], sem_s, add=True)
                       for j in range(nj)]
                v1c.wait()
                for cp in i1s:
                    cp.wait()
                for cp in a0s:
                    cp.wait()
                a1s = [pltpu.async_copy(rows_b[1].at[pl.ds(j * w, w)],
                                        acc.at[idx_b[1][j]], sem_s, add=True)
                       for j in range(nj)]
                for cp in a1s:
                    cp.wait()
                return carry

            lax.fori_loop(0, pairs, pair_body, 0)

        @pl.when(cid == 0)
        def _():
            stream(ilo)

        @pl.when(cid != 0)
        def _():
            stream(ihi)

        plsc.subcore_barrier()
        pltpu.sync_copy(acc.at[pl.ds(ob, per_sub)],
                        out.at[cid].at[pl.ds(ob, per_sub)])

    return k(vals, zeros, idx_lo, idx_hi)


def _segsum(vals, idx_lo, idx_hi, n):
    out = _sc_scatter_rs(vals, idx_lo, idx_hi)
    return jnp.concatenate([out[0, :HN], out[1, :n - HN]], axis=0)


NPAD = 10240


def _sc_scatter_two(vals, idxA, idxB):
    E = idxA.shape[0]
    w = IDXW
    ce = w
    pairs = E // NS // (2 * ce)
    per_e = E // NS
    per_sub = NPAD // NS
    mesh = plsc.VectorSubcoreMesh(core_axis_name="c", subcore_axis_name="s")
    zeros = jnp.zeros((per_sub, DH), F32)

    @functools.partial(
        pl.kernel,
        out_type=jax.ShapeDtypeStruct((NC, NPAD, DH), F32),
        mesh=mesh,
        scratch_types=[
            [pltpu.VMEM((w,), jnp.int32) for _ in range(2)],
            [pltpu.VMEM((ce, DH), F32) for _ in range(2)],
            pltpu.VMEM_SHARED((NPAD, DH), F32),
            pltpu.SemaphoreType.DMA,
            pltpu.SemaphoreType.DMA,
            pltpu.SemaphoreType.DMA,
        ],
    )
    def k(v, z, iA, iB, out, idx_b, rows_b, acc, sem_i, sem_v, sem_s):
        cid = lax.axis_index("c")
        sid = lax.axis_index("s")
        ob = sid * per_sub
        pltpu.sync_copy(z, acc.at[pl.ds(ob, per_sub)])
        plsc.subcore_barrier()

        def stream(idx):
            def pair_body(p, carry):
                b0 = sid * per_e + 2 * p * ce
                b1 = b0 + ce
                v0c = pltpu.async_copy(v.at[pl.ds(b0, ce)], rows_b[0], sem_v)
                i0c = pltpu.async_copy(idx.at[pl.ds(b0, w)], idx_b[0], sem_i)
                v1c = pltpu.async_copy(v.at[pl.ds(b1, ce)], rows_b[1], sem_v)
                i1c = pltpu.async_copy(idx.at[pl.ds(b1, w)], idx_b[1], sem_i)
                v0c.wait()
                i0c.wait()
                a0 = pltpu.async_copy(rows_b[0], acc.at[idx_b[0]], sem_s,
                                      add=True)
                v1c.wait()
                i1c.wait()
                a0.wait()
                a1 = pltpu.async_copy(rows_b[1], acc.at[idx_b[1]], sem_s,
                                      add=True)
                a1.wait()
                return carry

            lax.fori_loop(0, pairs, pair_body, 0)

        @pl.when(cid == 0)
        def _():
            stream(iA)

        @pl.when(cid != 0)
        def _():
            stream(iB)

        plsc.subcore_barrier()
        pltpu.sync_copy(acc.at[pl.ds(ob, per_sub)],
                        out.at[cid].at[pl.ds(ob, per_sub)])

    return k(vals, zeros, idxA, idxB)


def _enc_node_body(z, n, q16, W0z, W0n, b0, W1, b1, A0, B0,
                   xenc_o, sd_o, ts_o, td_o):
    pre = _mm(z[...], W0z[...]) + _mm(n[...], W0n[...]) + b0[...]
    h, sd = _silu_and_d(pre)
    sd_o[...] = sd
    xe = _mm(h, W1[...]) + b1[...]
    xenc_o[...] = xe
    qv = q16[...]
    pad = jnp.zeros((qv.shape[0], 112), F32)
    ts_o[...] = jnp.concatenate([_mm(xe, A0[...]), qv, pad], axis=1)
    td_o[...] = jnp.concatenate([_mm(xe, B0[...]), qv, pad], axis=1)


def _edge_fwd0_body(gs, gd, V0p, v0n, c0, V1, c1, C0, be0, W1, be1,
                    ea_o, sd_o, e1_o):
    d16 = gs[:, DH:DH + 16] - gd[:, DH:DH + 16]
    nrm = jnp.sqrt(jnp.sum(d16 * d16, axis=1, keepdims=True))
    pre0 = _mm(d16, V0p[...]) + _r16(nrm) * _r16(v0n[...]) + c0[...]
    h0, _ = _silu_and_d(pre0)
    e0 = _mm(h0, V1[...]) + c1[...]
    pre = _mm(e0, C0[...]) + gs[:, :DH] + gd[:, :DH] + be0[...]
    h, sd = _silu_and_d(pre)
    sd_o[...] = sd
    ea = _mm(h, W1[...]) + be1[...]
    ea_o[...] = ea
    e1_o[...] = e0 + ea


def _edge_fwd1_body(e, gs, gd, C, be0, W1, be1, ea_o, sd_o):
    pre = _mm(e[...], C[...]) + gs[...] + gd[...] + be0[...]
    h, sd = _silu_and_d(pre)
    sd_o[...] = sd
    ea_o[...] = _mm(h, W1[...]) + be1[...]


def _node_fwd_body(with_tables, x, agg, Wa, Wb, bn0, W1, bn1, *rest):
    if with_tables:
        A, B, xn_o, sd_o, xs_o, xd_o = rest
    else:
        xn_o, sd_o = rest
    pre = _mm(x[...], Wa[...]) + _mm(agg[...], Wb[...]) + bn0[...]
    h, sd = _silu_and_d(pre)
    sd_o[...] = sd
    xn = x[...] + _mm(h, W1[...]) + bn1[...]
    xn_o[...] = xn
    if with_tables:
        xs_o[...] = _mm(xn, A[...])
        xd_o[...] = _mm(xn, B[...])


_MMT_TERMS = None


def _mmt_terms():
    global _MMT_TERMS
    if _MMT_TERMS is None:
        pos = {}
        j = 0
        for a in range(6):
            for b in range(a + 1):
                pos[(a, b)] = j
                j += 1
        terms = []
        for a in range(6):
            for b in range(6):
                t = [(pos[(a, c)], pos[(b, c)]) for c in range(min(a, b) + 1)]
                terms.append(t)
        _MMT_TERMS = terms
    return _MMT_TERMS


def _dec_body(x, WE0, bE0, wE1, bE1, WS0, bS0, wS1, bS1,
              WL0, bL0, WL1f, bL1f, WM0, bM0, WM1, bM1,
              WE0T, WS0T,
              es_o, gxE_o, gxS_o, lcols_o, mmt_o):
    xv = x[...]
    preE = _mm(xv, WE0[...]) + bE0[...]
    hE, sdE = _silu_and_d(preE)
    Ev = jnp.sum(_r16(hE) * _r16(wE1[...]), axis=1, keepdims=True) + bE1[...]
    gxE_o[...] = _mm(sdE * _r16(wE1[...]), WE0T[...])
    preS = _mm(xv, WS0[...]) + bS0[...]
    hS, sdS = _silu_and_d(preS)
    Sv = jnp.sum(_r16(hS) * _r16(wS1[...]), axis=1, keepdims=True) + bS1[...]
    gxS_o[...] = _mm(sdS * _r16(wS1[...]), WS0T[...])
    z1 = jnp.zeros_like(Ev)
    es_o[...] = jnp.concatenate([Ev, Sv, z1, z1, z1, z1, z1, z1], axis=1)
    preL = _mm(xv, WL0[...]) + bL0[...]
    hL, _ = _silu_and_d(preL)
    lcols_o[...] = _mm(hL, WL1f[...]) + bL1f[...]
    preM = _mm(xv, WM0[...]) + bM0[...]
    hM, _ = _silu_and_d(preM)
    m = _r16(_mm(hM, WM1[...]) + bM1[...])
    cols = []
    for t in _mmt_terms():
        acc = m[:, t[0][0]:t[0][0] + 1] * m[:, t[0][1]:t[0][1] + 1]
        for (pa, pb) in t[1:]:
            acc = acc + m[:, pa:pa + 1] * m[:, pb:pb + 1]
        cols.append(acc)
    mmt_o[...] = jnp.concatenate(cols, axis=1)


def _node_bwd1_body(gE, gS, sdn, Wn1T, WaT, WbT,
                    gaE_o, gaS_o, gpE_o, gpS_o):
    for g, ga_o, gp_o in ((gE, gaE_o, gpE_o), (gS, gaS_o, gpS_o)):
        gpren = _mm(g[...], Wn1T[...]) * sdn[...]
        ga_o[...] = _mm(gpren, WbT[...])
        gp_o[...] = g[...] + _mm(gpren, WaT[...])


def _edge_bwd1_body(ggE, ggS, sd, We1T, CT,
                    gprE_o, gprS_o, geE_o, geS_o):
    for gg, gpr_o, ge_o in ((ggE, gprE_o, geE_o), (ggS, gprS_o, geS_o)):
        gpr = _mm(gg[...], We1T[...]) * sd[...]
        gpr_o[...] = gpr
        ge_o[...] = _mm(gpr, CT[...])


def _edge_bwd0_body(ggE, ggS, geE, geS, sd, We1T, gprE_o, gprS_o):
    for gg, ge, gpr_o in ((ggE, geE, gprE_o), (ggS, geS, gprS_o)):
        gpr_o[...] = _mm(gg[...] + ge[...], We1T[...]) * sd[...]


def _node_bwd_mid_body(gpE, gpS, ssE, sdE, ssS, sdS, sdn0,
                       A1T, B1T, Wn1T0, WaT0, WbT0,
                       gaE_o, gaS_o, gpE_o, gpS_o):
    for gp, ss, sdst, ga_o, gp_o in (
            (gpE, ssE, sdE, gaE_o, gpE_o), (gpS, ssS, sdS, gaS_o, gpS_o)):
        g1 = gp[...] + _mm(ss[...], A1T[...]) + _mm(sdst[...], B1T[...])
        gpren = _mm(g1, Wn1T0[...]) * sdn0[...]
        ga_o[...] = _mm(gpren, WbT0[...])
        gp_o[...] = g1 + _mm(gpren, WaT0[...])


def _node_bwd_final_body(gpE, gpS, ssE, sdE, ssS, sdS, sdenc,
                         A0T, B0T, W1eT, W0eT, dzE_o, dzS_o):
    for gp, ss, sdst, dz_o in (
            (gpE, ssE, sdE, dzE_o), (gpS, ssS, sdS, dzS_o)):
        g0 = gp[...] + _mm(ss[...], A0T[...]) + _mm(sdst[...], B0T[...])
        gpre = _mm(g0, W1eT[...]) * sdenc[...]
        dz_o[...] = _mm(gpre, W0eT[...])


def kernel(z, n, edge_index, q_0, params):
    N = z.shape[0]
    E = edge_index.shape[1]
    src1 = edge_index[0]
    dest1 = edge_index[1]
    s_lo = jnp.where(src1 < HN, src1, HN)
    s_hi = jnp.where(src1 >= HN, src1 - HN, HN)
    d_lo = jnp.where(dest1 < HN, dest1, HN)
    d_hi = jnp.where(dest1 >= HN, dest1 - HN, HN)

    (W0e, b0e), (W1e, b1e) = params["enc_node"]
    (V0, c0), (V1, c1) = params["enc_edge"]
    em = params["edge_mlps"]
    nm = params["node_mlps"]
    C = [em[p][0][0][:DH] for p in range(2)]
    A = [em[p][0][0][DH:2 * DH] for p in range(2)]
    B = [em[p][0][0][2 * DH:] for p in range(2)]
    be0 = [em[p][0][1].reshape(1, -1) for p in range(2)]
    We1 = [em[p][1][0] for p in range(2)]
    be1 = [em[p][1][1].reshape(1, -1) for p in range(2)]
    Wa = [nm[p][0][0][:DH] for p in range(2)]
    Wb = [nm[p][0][0][DH:] for p in range(2)]
    bn0 = [nm[p][0][1].reshape(1, -1) for p in range(2)]
    Wn1 = [nm[p][1][0] for p in range(2)]
    bn1 = [nm[p][1][1].reshape(1, -1) for p in range(2)]

    q16 = jnp.pad(q_0, ((0, 0), (0, 13)))
    xenc, sdenc, tbl_s, tbl_d = _tc_call(
        _enc_node_body, N, NB,
        [z, n, q16],
        [W0e[:6], W0e[6:], b0e.reshape(1, -1), W1e, b1e.reshape(1, -1),
         A[0], B[0]],
        [DH, DH, 256, 256], "enc_node")

    gs0, gd0 = _sc_gather2(tbl_s, tbl_d, src1, dest1, 256)

    V0p = jnp.pad(V0[:3], ((0, 13), (0, 0)))
    ea0, sde0, e1 = _tc_call(
        _edge_fwd0_body, E, EB,
        [gs0, gd0],
        [V0p, V0[3].reshape(1, -1), c0.reshape(1, -1), V1, c1.reshape(1, -1),
         C[0], be0[0], We1[0], be1[0]],
        [DH, DH, DH], "edge_fwd0")

    agg0 = _segsum(ea0, d_lo, d_hi, N)
    x1, sdn0, xs1, xd1 = _tc_call(
        functools.partial(_node_fwd_body, True), N, NB,
        [xenc, agg0],
        [Wa[0], Wb[0], bn0[0], Wn1[0], bn1[0], A[1], B[1]],
        [DH, DH, DH, DH], "node_fwd0")

    gxs1, gxd1 = _sc_gather2_staged(xs1, xd1, src1, dest1)
    ea1, sde1 = _tc_call(
        _edge_fwd1_body, E, EB,
        [e1, gxs1, gxd1],
        [C[1], be0[1], We1[1], be1[1]],
        [DH, DH], "edge_fwd1")

    agg1 = _segsum(ea1, d_lo, d_hi, N)
    x2, sdn1 = _tc_call(
        functools.partial(_node_fwd_body, False), N, NB,
        [x1, agg1],
        [Wa[1], Wb[1], bn0[1], Wn1[1], bn1[1]],
        [DH, DH], "node_fwd1")

    (WE0, bE0), (WE1, bE1) = params["dec_E"]
    (WS0, bS0), (WS1, bS1) = params["dec_S"]
    (WL0, bL0), (WL1, bL1) = params["dec_L"]
    (WM0, bM0), (WM1, bM1) = params["dec_M"]

    import numpy as _np
    rL, cL = _np.tril_indices(6, -1)
    PL = _np.zeros((15, 36), _np.float32)
    for j, (a, b) in enumerate(zip(rL, cL)):
        PL[j, a * 6 + b] = 1.0
        PL[j, b * 6 + a] = -1.0
    PLj = jnp.asarray(PL)
    WL1f = WL1 @ PLj
    bL1f = (bL1 @ PLj).reshape(1, -1)

    es8, gxE, gxS, lcols, mmt = _tc_call(
        _dec_body, N, NB,
        [x2],
        [WE0, bE0.reshape(1, -1), WE1[:, 0].reshape(1, -1), bE1.reshape(1, -1),
         WS0, bS0.reshape(1, -1), WS1[:, 0].reshape(1, -1), bS1.reshape(1, -1),
         WL0, bL0.reshape(1, -1), WL1f, bL1f,
         WM0, bM0.reshape(1, -1), WM1, bM1.reshape(1, -1),
         WE0.T, WS0.T],
        [8, DH, DH, 36, 36], "dec")

    gaE1, gaS1, gpE1, gpS1 = _tc_call(
        _node_bwd1_body, N, NB,
        [gxE, gxS, sdn1],
        [Wn1[1].T, Wa[1].T, Wb[1].T],
        [DH, DH, DH, DH], "node_bwd1")

    ggE1, ggS1 = _sc_gather2_staged(gaE1, gaS1, dest1, dest1)
    gprE1, gprS1, geE, geS = _tc_call(
        _edge_bwd1_body, E, EB,
        [ggE1, ggS1, sde1],
        [We1[1].T, C[1].T],
        [DH, DH, DH, DH], "edge_bwd1")

    sE1 = _sc_scatter_two(gprE1, src1, dest1)
    sS1 = _sc_scatter_two(gprS1, src1, dest1)
    ssE1, sdE1 = sE1[0, :N], sE1[1, :N]
    ssS1, sdS1 = sS1[0, :N], sS1[1, :N]

    gaE0, gaS0, gpE0, gpS0 = _tc_call(
        _node_bwd_mid_body, N, NB,
        [gpE1, gpS1, ssE1, sdE1, ssS1, sdS1, sdn0],
        [A[1].T, B[1].T, Wn1[0].T, Wa[0].T, Wb[0].T],
        [DH, DH, DH, DH], "node_bwd_mid")

    ggE0, ggS0 = _sc_gather2_staged(gaE0, gaS0, dest1, dest1)
    gprE0, gprS0 = _tc_call(
        _edge_bwd0_body, E, EB,
        [ggE0, ggS0, geE, geS, sde0],
        [We1[0].T],
        [DH, DH], "edge_bwd0")

    sE0 = _sc_scatter_two(gprE0, src1, dest1)
    sS0 = _sc_scatter_two(gprS0, src1, dest1)
    ssE0, sdE0 = sE0[0, :N], sE0[1, :N]
    ssS0, sdS0 = sS0[0, :N], sS0[1, :N]

    dzE8, dzS8 = _tc_call(
        _node_bwd_final_body, N, NB,
        [gpE0, gpS0, ssE0, sdE0, ssS0, sdS0, sdenc],
        [A[0].T, B[0].T, W1e.T, W0e.T],
        [8, 8], "node_bwd_final")

    L = lcols.reshape(N, 6, 6)
    M = mmt.reshape(N, 6, 6)
    dEdz = dzE8[:, :6].reshape(N, 6, 1)
    dSdz = dzS8[:, :6].reshape(N, 6, 1)
    E_out = es8[:, 0:1]
    S_out = es8[:, 1:2]
    return (L, M, dEdz, dSdz, E_out, S_out)
